# Initial kernel scaffold; baseline (speedup 1.0000x reference)
import jax
import jax.numpy as jnp
from jax.experimental import pallas as pl


def kernel(x, edge_index, W, b, prelu_a):
    raise NotImplementedError("write your pallas kernel here")


# speedup vs baseline: 17.0684x; 17.0684x over previous
import functools

import jax
import jax.numpy as jnp
from jax import lax
from jax.experimental import pallas as pl
from jax.experimental.pallas import tpu as pltpu
from jax.experimental.pallas import tpu_sc as plsc

N = 10000
E = 320000
D = 128
NC = 2
NS = 16
NW = NC * NS
EPW = E // NW
CHUNK = 80
NCHUNK = EPW // CHUNK
NPAD = 10240
RPS = NPAD // NS
DEGW = 16
L = 16

_MESH = dict(core_axis_name="c", subcore_axis_name="s", num_cores=NC,
             num_subcores=NS)


def _zero_vmem_2d(ref, rows, cols):
    z = jnp.zeros((L,), jnp.float32)

    def body(k, _):
        i = k // (cols // L)
        j = k % (cols // L)
        ref[i, pl.ds(j * L, L)] = z
        return _

    lax.fori_loop(0, rows * (cols // L), body, None)


@functools.partial(
    pl.kernel,
    out_type=jax.ShapeDtypeStruct((NC, NPAD, DEGW), jnp.float32),
    mesh=plsc.VectorSubcoreMesh(**_MESH),
    scratch_types=[
        pltpu.VMEM((CHUNK,), jnp.int32),
        pltpu.VMEM((CHUNK, DEGW), jnp.float32),
        pltpu.VMEM_SHARED((NPAD, DEGW), jnp.float32),
    ],
)
def _sc_degree(dst_hbm, out_hbm, didx_v, ones_v, acc_sh):
    c = lax.axis_index("c")
    s = lax.axis_index("s")
    wid = c * NS + s

    _zero_vmem_2d(ones_v, CHUNK, DEGW)

    def zinit(j, _):
        pltpu.sync_copy(ones_v, acc_sh.at[pl.ds(s * RPS + j * CHUNK, CHUNK)])
        return _

    lax.fori_loop(0, RPS // CHUNK, zinit, None)

    one = jnp.full((L,), 1.0, jnp.float32)

    def fill(i, _):
        ones_v[i, :] = one
        return _

    lax.fori_loop(0, CHUNK, fill, None)
    plsc.subcore_barrier()

    def step(i, _):
        base = wid * EPW + i * CHUNK
        pltpu.sync_copy(dst_hbm.at[pl.ds(base, CHUNK)], didx_v)
        pltpu.sync_copy(ones_v, acc_sh.at[didx_v], add=True)
        return _

    lax.fori_loop(0, NCHUNK, step, None)
    plsc.subcore_barrier()

    def wb(j, _):
        r0 = s * RPS + j * CHUNK
        pltpu.sync_copy(acc_sh.at[pl.ds(r0, CHUNK)], ones_v)
        pltpu.sync_copy(ones_v, out_hbm.at[c, pl.ds(r0, CHUNK)])
        return _

    lax.fori_loop(0, RPS // CHUNK, wb, None)


_RB = 2000


def _prep_body(x_ref, w_ref, dp_ref, g_ref):
    h = jnp.dot(x_ref[...], w_ref[...], preferred_element_type=jnp.float32)
    deg = dp_ref[0, :, 0:1] + dp_ref[1, :, 0:1] + 1.0
    g_ref[...] = h * lax.rsqrt(deg)


def _tc_prep(x, W, degpart):
    return pl.pallas_call(
        _prep_body,
        grid=(N // _RB,),
        in_specs=[
            pl.BlockSpec((_RB, D), lambda i: (i, 0)),
            pl.BlockSpec((D, D), lambda i: (0, 0)),
            pl.BlockSpec((NC, _RB, DEGW), lambda i: (0, i, 0)),
        ],
        out_specs=pl.BlockSpec((_RB, D), lambda i: (i, 0)),
        out_shape=jax.ShapeDtypeStruct((N, D), jnp.float32),
    )(x, W, degpart)


@functools.partial(
    pl.kernel,
    out_type=jax.ShapeDtypeStruct((NC, NPAD, D), jnp.float32),
    mesh=plsc.VectorSubcoreMesh(**_MESH),
    scratch_types=[
        pltpu.VMEM((CHUNK,), jnp.int32),
        pltpu.VMEM((CHUNK,), jnp.int32),
        pltpu.VMEM((CHUNK, D), jnp.float32),
        pltpu.VMEM_SHARED((NPAD, D), jnp.float32),
    ],
)
def _sc_edges(g_hbm, src_hbm, dst_hbm, out_hbm, sidx_v, didx_v, rows_v,
              acc_sh):
    c = lax.axis_index("c")
    s = lax.axis_index("s")
    wid = c * NS + s

    _zero_vmem_2d(rows_v, CHUNK, D)

    def zinit(j, _):
        pltpu.sync_copy(rows_v, acc_sh.at[pl.ds(s * RPS + j * CHUNK, CHUNK)])
        return _

    lax.fori_loop(0, RPS // CHUNK, zinit, None)
    plsc.subcore_barrier()

    def step(i, _):
        base = wid * EPW + i * CHUNK
        pltpu.sync_copy(src_hbm.at[pl.ds(base, CHUNK)], sidx_v)
        pltpu.sync_copy(dst_hbm.at[pl.ds(base, CHUNK)], didx_v)
        pltpu.sync_copy(g_hbm.at[sidx_v], rows_v)
        pltpu.sync_copy(rows_v, acc_sh.at[didx_v], add=True)
        return _

    lax.fori_loop(0, NCHUNK, step, None)
    plsc.subcore_barrier()

    def wb(j, _):
        r0 = s * RPS + j * CHUNK
        pltpu.sync_copy(acc_sh.at[pl.ds(r0, CHUNK)], rows_v)
        pltpu.sync_copy(rows_v, out_hbm.at[c, pl.ds(r0, CHUNK)])
        return _

    lax.fori_loop(0, RPS // CHUNK, wb, None)


def _final_body(ap_ref, g_ref, dp_ref, b_ref, a_ref, o_ref):
    deg = dp_ref[0, :, 0:1] + dp_ref[1, :, 0:1] + 1.0
    dinv = lax.rsqrt(deg)
    z = dinv * (ap_ref[0] + ap_ref[1] + g_ref[...]) + b_ref[...]
    o_ref[...] = jnp.where(z >= 0, z, a_ref[0, 0] * z)


def _tc_final(accpart, g, degpart, b2, a2):
    return pl.pallas_call(
        _final_body,
        grid=(N // _RB,),
        in_specs=[
            pl.BlockSpec((NC, _RB, D), lambda i: (0, i, 0)),
            pl.BlockSpec((_RB, D), lambda i: (i, 0)),
            pl.BlockSpec((NC, _RB, DEGW), lambda i: (0, i, 0)),
            pl.BlockSpec((1, D), lambda i: (0, 0)),
            pl.BlockSpec((1, 1), lambda i: (0, 0)),
        ],
        out_specs=pl.BlockSpec((_RB, D), lambda i: (i, 0)),
        out_shape=jax.ShapeDtypeStruct((N, D), jnp.float32),
    )(accpart, g, degpart, b2, a2)


def kernel(x, edge_index, W, b, prelu_a):
    ei = edge_index.astype(jnp.int32)
    src = ei[0]
    dst = ei[1]
    degpart = _sc_degree(dst)[:, :N]
    g = _tc_prep(x, W, degpart)
    accpart = _sc_edges(g, src, dst)[:, :N]
    return _tc_final(accpart, g, degpart, b.reshape(1, D),
                     prelu_a.reshape(1, 1))


# speedup vs baseline: 40.3040x; 2.3613x over previous
import functools

import jax
import jax.numpy as jnp
from jax import lax
from jax.experimental import pallas as pl
from jax.experimental.pallas import tpu as pltpu
from jax.experimental.pallas import tpu_sc as plsc

N = 10000
E = 320000
D = 128
NC = 2
NS = 16
NW = NC * NS
EPW = E // NW
CHUNK = 80
NCHUNK = EPW // CHUNK
NPAD = 10240
RPS = NPAD // NS
DEGW = 16
L = 16
NBUF = 3

_MESH = dict(core_axis_name="c", subcore_axis_name="s", num_cores=NC,
             num_subcores=NS)


def _zero_vmem_2d(ref, rows, cols):
    z = jnp.zeros((L,), jnp.float32)

    def body(k, _):
        i = k // (cols // L)
        j = k % (cols // L)
        ref[i, pl.ds(j * L, L)] = z
        return _

    lax.fori_loop(0, rows * (cols // L), body, None)


@functools.partial(
    pl.kernel,
    out_type=jax.ShapeDtypeStruct((NC, NPAD, DEGW), jnp.float32),
    mesh=plsc.VectorSubcoreMesh(**_MESH),
    scratch_types=[
        pltpu.VMEM((NBUF, CHUNK), jnp.int32),
        pltpu.VMEM((CHUNK, DEGW), jnp.float32),
        pltpu.VMEM_SHARED((NPAD, DEGW), jnp.float32),
        pltpu.SemaphoreType.DMA((NBUF,)),
        pltpu.SemaphoreType.DMA((NBUF,)),
    ],
)
def _sc_degree(dst_hbm, out_hbm, didx_v, ones_v, acc_sh, dsem, ssem):
    c = lax.axis_index("c")
    s = lax.axis_index("s")
    wid = c * NS + s

    _zero_vmem_2d(ones_v, CHUNK, DEGW)

    def zinit(j, _):
        pltpu.sync_copy(ones_v, acc_sh.at[pl.ds(s * RPS + j * CHUNK, CHUNK)])
        return _

    lax.fori_loop(0, RPS // CHUNK, zinit, None)

    one = jnp.full((L,), 1.0, jnp.float32)

    def fill(i, _):
        ones_v[i, :] = one
        return _

    lax.fori_loop(0, CHUNK, fill, None)
    plsc.subcore_barrier()

    def didx_copy(i, b):
        base = wid * EPW + i * CHUNK
        return pltpu.make_async_copy(dst_hbm.at[pl.ds(base, CHUNK)],
                                     didx_v.at[b], dsem.at[b])

    def scat_wait(b):
        pltpu.make_async_copy(ones_v, acc_sh.at[didx_v.at[b]],
                              ssem.at[b]).wait()

    didx_copy(0, 0).start()

    def step(i, _):
        b = lax.rem(i, NBUF)
        b1 = lax.rem(i + 1, NBUF)

        @pl.when(i + 1 < NCHUNK)
        def _():
            @pl.when(i + 1 >= NBUF)
            def _():
                scat_wait(b1)
            didx_copy(i + 1, b1).start()

        didx_copy(i, b).wait()
        pltpu.async_copy(ones_v, acc_sh.at[didx_v.at[b]], ssem.at[b],
                         add=True)
        return _

    lax.fori_loop(0, NCHUNK, step, None)
    for t in range(NBUF):
        scat_wait((NCHUNK - 1 - t) % NBUF)
    plsc.subcore_barrier()

    def wb(j, _):
        r0 = s * RPS + j * CHUNK
        pltpu.sync_copy(acc_sh.at[pl.ds(r0, CHUNK)], ones_v)
        pltpu.sync_copy(ones_v, out_hbm.at[c, pl.ds(r0, CHUNK)])
        return _

    lax.fori_loop(0, RPS // CHUNK, wb, None)


_RB = 2000


def _prep_body(x_ref, w_ref, dp_ref, g_ref):
    h = jnp.dot(x_ref[...], w_ref[...], preferred_element_type=jnp.float32)
    deg = dp_ref[0, :, 0:1] + dp_ref[1, :, 0:1] + 1.0
    g_ref[...] = h * lax.rsqrt(deg)


def _tc_prep(x, W, degpart):
    return pl.pallas_call(
        _prep_body,
        grid=(N // _RB,),
        in_specs=[
            pl.BlockSpec((_RB, D), lambda i: (i, 0)),
            pl.BlockSpec((D, D), lambda i: (0, 0)),
            pl.BlockSpec((NC, _RB, DEGW), lambda i: (0, i, 0)),
        ],
        out_specs=pl.BlockSpec((_RB, D), lambda i: (i, 0)),
        out_shape=jax.ShapeDtypeStruct((N, D), jnp.float32),
    )(x, W, degpart)


@functools.partial(
    pl.kernel,
    out_type=jax.ShapeDtypeStruct((NC, NPAD, D), jnp.float32),
    mesh=plsc.VectorSubcoreMesh(**_MESH),
    scratch_types=[
        pltpu.VMEM((EPW,), jnp.int32),
        pltpu.VMEM((NBUF, CHUNK), jnp.int32),
        pltpu.VMEM((NBUF, CHUNK, D), jnp.float32),
        pltpu.VMEM_SHARED((NPAD, D), jnp.float32),
        pltpu.SemaphoreType.DMA((NBUF,)),
        pltpu.SemaphoreType.DMA((NBUF,)),
        pltpu.SemaphoreType.DMA((NBUF,)),
    ],
)
def _sc_edges(g_hbm, src_hbm, dst_hbm, out_hbm, sidx_v, didx_v, rows_v,
              acc_sh, dsem, gsem, ssem):
    c = lax.axis_index("c")
    s = lax.axis_index("s")
    wid = c * NS + s

    _zero_vmem_2d(rows_v.at[0], CHUNK, D)

    def zinit(j, _):
        pltpu.sync_copy(rows_v.at[0],
                        acc_sh.at[pl.ds(s * RPS + j * CHUNK, CHUNK)])
        return _

    lax.fori_loop(0, RPS // CHUNK, zinit, None)
    pltpu.sync_copy(src_hbm.at[pl.ds(wid * EPW, EPW)], sidx_v)
    plsc.subcore_barrier()

    def didx_copy(i, b):
        base = wid * EPW + i * CHUNK
        return pltpu.make_async_copy(dst_hbm.at[pl.ds(base, CHUNK)],
                                     didx_v.at[b], dsem.at[b])

    def gath(i, b):
        return pltpu.make_async_copy(
            g_hbm.at[sidx_v.at[pl.ds(i * CHUNK, CHUNK)]], rows_v.at[b],
            gsem.at[b])

    def scat_wait(b):
        pltpu.make_async_copy(rows_v.at[b], acc_sh.at[didx_v.at[b]],
                              ssem.at[b]).wait()

    didx_copy(0, 0).start()
    gath(0, 0).start()

    def step(i, _):
        b = lax.rem(i, NBUF)
        b1 = lax.rem(i + 1, NBUF)

        @pl.when(i + 1 < NCHUNK)
        def _():
            @pl.when(i + 1 >= NBUF)
            def _():
                scat_wait(b1)
            didx_copy(i + 1, b1).start()
            gath(i + 1, b1).start()

        didx_copy(i, b).wait()
        gath(i, b).wait()
        pltpu.async_copy(rows_v.at[b], acc_sh.at[didx_v.at[b]], ssem.at[b],
                         add=True)
        return _

    lax.fori_loop(0, NCHUNK, step, None)
    for t in range(NBUF):
        scat_wait((NCHUNK - 1 - t) % NBUF)
    plsc.subcore_barrier()

    def wb(j, _):
        r0 = s * RPS + j * CHUNK
        pltpu.sync_copy(acc_sh.at[pl.ds(r0, CHUNK)], rows_v.at[0])
        pltpu.sync_copy(rows_v.at[0], out_hbm.at[c, pl.ds(r0, CHUNK)])
        return _

    lax.fori_loop(0, RPS // CHUNK, wb, None)


def _final_body(ap_ref, g_ref, dp_ref, b_ref, a_ref, o_ref):
    deg = dp_ref[0, :, 0:1] + dp_ref[1, :, 0:1] + 1.0
    dinv = lax.rsqrt(deg)
    z = dinv * (ap_ref[0] + ap_ref[1] + g_ref[...]) + b_ref[...]
    o_ref[...] = jnp.where(z >= 0, z, a_ref[0, 0] * z)


def _tc_final(accpart, g, degpart, b2, a2):
    return pl.pallas_call(
        _final_body,
        grid=(N // _RB,),
        in_specs=[
            pl.BlockSpec((NC, _RB, D), lambda i: (0, i, 0)),
            pl.BlockSpec((_RB, D), lambda i: (i, 0)),
            pl.BlockSpec((NC, _RB, DEGW), lambda i: (0, i, 0)),
            pl.BlockSpec((1, D), lambda i: (0, 0)),
            pl.BlockSpec((1, 1), lambda i: (0, 0)),
        ],
        out_specs=pl.BlockSpec((_RB, D), lambda i: (i, 0)),
        out_shape=jax.ShapeDtypeStruct((N, D), jnp.float32),
    )(accpart, g, degpart, b2, a2)


def kernel(x, edge_index, W, b, prelu_a):
    ei = edge_index.astype(jnp.int32)
    src = ei[0]
    dst = ei[1]
    degpart = _sc_degree(dst)
    g = _tc_prep(x, W, degpart)
    accpart = _sc_edges(g, src, dst)
    return _tc_final(accpart, g, degpart, b.reshape(1, D),
                     prelu_a.reshape(1, 1))


# speedup vs baseline: 44.7709x; 1.1108x over previous
import functools

import jax
import jax.numpy as jnp
from jax import lax
from jax.experimental import pallas as pl
from jax.experimental.pallas import tpu as pltpu
from jax.experimental.pallas import tpu_sc as plsc

N = 10000
E = 320000
D = 128
NC = 2
NS = 16
NW = NC * NS
EPW = E // NW
CHUNK = 80
NCHUNK = EPW // CHUNK
NPAD = 10240
RPS = NPAD // NS
DEGW = 16
L = 16
NBUF = 3

_MESH = dict(core_axis_name="c", subcore_axis_name="s", num_cores=NC,
             num_subcores=NS)


def _zero_vmem_2d(ref, rows, cols):
    z = jnp.zeros((L,), jnp.float32)

    def body(k, _):
        i = k // (cols // L)
        j = k % (cols // L)
        ref[i, pl.ds(j * L, L)] = z
        return _

    lax.fori_loop(0, rows * (cols // L), body, None)


@functools.partial(
    pl.kernel,
    out_type=jax.ShapeDtypeStruct((NC, NPAD, DEGW), jnp.float32),
    mesh=plsc.VectorSubcoreMesh(**_MESH),
    scratch_types=[
        pltpu.VMEM((NCHUNK, CHUNK), jnp.int32),
        pltpu.VMEM((CHUNK, DEGW), jnp.float32),
        pltpu.VMEM_SHARED((NPAD, DEGW), jnp.float32),
        pltpu.SemaphoreType.DMA((NBUF,)),
        pltpu.SemaphoreType.DMA,
    ],
)
def _sc_degree(dst_hbm, out_hbm, didx_v, ones_v, acc_sh, ssem, psem):
    c = lax.axis_index("c")
    s = lax.axis_index("s")
    wid = c * NS + s

    _zero_vmem_2d(ones_v, CHUNK, DEGW)

    def zinit(j, _):
        pltpu.sync_copy(ones_v, acc_sh.at[pl.ds(s * RPS + j * CHUNK, CHUNK)])
        return _

    lax.fori_loop(0, RPS // CHUNK, zinit, None)

    one = jnp.full((L,), 1.0, jnp.float32)

    def fill(i, _):
        ones_v[i, :] = one
        return _

    lax.fori_loop(0, CHUNK, fill, None)

    def pre(i, _):
        base = wid * EPW + i * CHUNK
        pltpu.make_async_copy(dst_hbm.at[pl.ds(base, CHUNK)], didx_v.at[i],
                              psem).start()
        return _

    def pre_wait(i, _):
        pltpu.make_async_copy(dst_hbm.at[pl.ds(wid * EPW, CHUNK)],
                              didx_v.at[0], psem).wait()
        return _

    lax.fori_loop(0, NCHUNK, pre, None)
    lax.fori_loop(0, NCHUNK, pre_wait, None)
    plsc.subcore_barrier()

    def scat_wait(b):
        pltpu.make_async_copy(ones_v, acc_sh.at[didx_v.at[0]],
                              ssem.at[b]).wait()

    def step(i, _):
        b = lax.rem(i, NBUF)

        @pl.when(i >= NBUF)
        def _():
            scat_wait(b)

        pltpu.async_copy(ones_v, acc_sh.at[didx_v.at[i]], ssem.at[b],
                         add=True)
        return _

    lax.fori_loop(0, NCHUNK, step, None)
    for t in range(NBUF):
        scat_wait((NCHUNK - 1 - t) % NBUF)
    plsc.subcore_barrier()

    def wb(j, _):
        r0 = s * RPS + j * CHUNK
        pltpu.sync_copy(acc_sh.at[pl.ds(r0, CHUNK)], ones_v)
        pltpu.sync_copy(ones_v, out_hbm.at[c, pl.ds(r0, CHUNK)])
        return _

    lax.fori_loop(0, RPS // CHUNK, wb, None)


_RB = 2000


def _prep_body(x_ref, w_ref, dp_ref, g_ref):
    h = jnp.dot(x_ref[...], w_ref[...], preferred_element_type=jnp.float32)
    deg = dp_ref[0, :, 0:1] + dp_ref[1, :, 0:1] + 1.0
    g_ref[...] = h * lax.rsqrt(deg)


def _tc_prep(x, W, degpart):
    return pl.pallas_call(
        _prep_body,
        grid=(N // _RB,),
        in_specs=[
            pl.BlockSpec((_RB, D), lambda i: (i, 0)),
            pl.BlockSpec((D, D), lambda i: (0, 0)),
            pl.BlockSpec((NC, _RB, DEGW), lambda i: (0, i, 0)),
        ],
        out_specs=pl.BlockSpec((_RB, D), lambda i: (i, 0)),
        out_shape=jax.ShapeDtypeStruct((N, D), jnp.float32),
    )(x, W, degpart)


@functools.partial(
    pl.kernel,
    out_type=jax.ShapeDtypeStruct((NC, NPAD, D), jnp.float32),
    mesh=plsc.VectorSubcoreMesh(**_MESH),
    scratch_types=[
        pltpu.VMEM((EPW,), jnp.int32),
        pltpu.VMEM((NBUF, CHUNK), jnp.int32),
        pltpu.VMEM((NBUF, CHUNK, D), jnp.float32),
        pltpu.VMEM_SHARED((NPAD, D), jnp.float32),
        pltpu.SemaphoreType.DMA((NBUF,)),
        pltpu.SemaphoreType.DMA((NBUF,)),
        pltpu.SemaphoreType.DMA((NBUF,)),
    ],
)
def _sc_edges(g_hbm, src_hbm, dst_hbm, out_hbm, sidx_v, didx_v, rows_v,
              acc_sh, dsem, gsem, ssem):
    c = lax.axis_index("c")
    s = lax.axis_index("s")
    wid = c * NS + s

    _zero_vmem_2d(rows_v.at[0], CHUNK, D)

    def zinit(j, _):
        pltpu.sync_copy(rows_v.at[0],
                        acc_sh.at[pl.ds(s * RPS + j * CHUNK, CHUNK)])
        return _

    lax.fori_loop(0, RPS // CHUNK, zinit, None)
    pltpu.sync_copy(src_hbm.at[pl.ds(wid * EPW, EPW)], sidx_v)
    plsc.subcore_barrier()

    def didx_copy(i, b):
        base = wid * EPW + i * CHUNK
        return pltpu.make_async_copy(dst_hbm.at[pl.ds(base, CHUNK)],
                                     didx_v.at[b], dsem.at[b])

    def gath(i, b):
        return pltpu.make_async_copy(
            g_hbm.at[sidx_v.at[pl.ds(i * CHUNK, CHUNK)]], rows_v.at[b],
            gsem.at[b])

    def scat_wait(b):
        pltpu.make_async_copy(rows_v.at[b], acc_sh.at[didx_v.at[b]],
                              ssem.at[b]).wait()

    didx_copy(0, 0).start()
    gath(0, 0).start()

    def step(i, _):
        b = lax.rem(i, NBUF)
        b1 = lax.rem(i + 1, NBUF)

        @pl.when(i + 1 < NCHUNK)
        def _():
            @pl.when(i + 1 >= NBUF)
            def _():
                scat_wait(b1)
            didx_copy(i + 1, b1).start()
            gath(i + 1, b1).start()

        didx_copy(i, b).wait()
        gath(i, b).wait()
        pltpu.async_copy(rows_v.at[b], acc_sh.at[didx_v.at[b]], ssem.at[b],
                         add=True)
        return _

    lax.fori_loop(0, NCHUNK, step, None)
    for t in range(NBUF):
        scat_wait((NCHUNK - 1 - t) % NBUF)
    plsc.subcore_barrier()

    def wb(j, _):
        r0 = s * RPS + j * CHUNK
        pltpu.sync_copy(acc_sh.at[pl.ds(r0, CHUNK)], rows_v.at[0])
        pltpu.sync_copy(rows_v.at[0], out_hbm.at[c, pl.ds(r0, CHUNK)])
        return _

    lax.fori_loop(0, RPS // CHUNK, wb, None)


def _final_body(ap_ref, g_ref, dp_ref, b_ref, a_ref, o_ref):
    deg = dp_ref[0, :, 0:1] + dp_ref[1, :, 0:1] + 1.0
    dinv = lax.rsqrt(deg)
    z = dinv * (ap_ref[0] + ap_ref[1] + g_ref[...]) + b_ref[...]
    o_ref[...] = jnp.where(z >= 0, z, a_ref[0, 0] * z)


def _tc_final(accpart, g, degpart, b2, a2):
    return pl.pallas_call(
        _final_body,
        grid=(N // _RB,),
        in_specs=[
            pl.BlockSpec((NC, _RB, D), lambda i: (0, i, 0)),
            pl.BlockSpec((_RB, D), lambda i: (i, 0)),
            pl.BlockSpec((NC, _RB, DEGW), lambda i: (0, i, 0)),
            pl.BlockSpec((1, D), lambda i: (0, 0)),
            pl.BlockSpec((1, 1), lambda i: (0, 0)),
        ],
        out_specs=pl.BlockSpec((_RB, D), lambda i: (i, 0)),
        out_shape=jax.ShapeDtypeStruct((N, D), jnp.float32),
    )(accpart, g, degpart, b2, a2)


def kernel(x, edge_index, W, b, prelu_a):
    ei = edge_index.astype(jnp.int32)
    src = ei[0]
    dst = ei[1]
    degpart = _sc_degree(dst)
    g = _tc_prep(x, W, degpart)
    accpart = _sc_edges(g, src, dst)
    return _tc_final(accpart, g, degpart, b.reshape(1, D),
                     prelu_a.reshape(1, 1))


# speedup vs baseline: 48.0560x; 1.0734x over previous
import functools

import jax
import jax.numpy as jnp
from jax import lax
from jax.experimental import pallas as pl
from jax.experimental.pallas import tpu as pltpu
from jax.experimental.pallas import tpu_sc as plsc

N = 10000
E = 320000
D = 128
NC = 2
NS = 16
NW = NC * NS
EPW = E // NW
CHUNK = 80
NCHUNK = EPW // CHUNK
NPAD = 10240
RPS = NPAD // NS
DEGW = 16
L = 16
NBUF = 3

_MESH = dict(core_axis_name="c", subcore_axis_name="s", num_cores=NC,
             num_subcores=NS)


def _zero_vmem_2d(ref, rows, cols):
    z = jnp.zeros((L,), jnp.float32)

    def body(k, _):
        i = k // (cols // L)
        j = k % (cols // L)
        ref[i, pl.ds(j * L, L)] = z
        return _

    lax.fori_loop(0, rows * (cols // L), body, None)


@functools.partial(
    pl.kernel,
    out_type=jax.ShapeDtypeStruct((NC, NPAD, DEGW), jnp.float32),
    mesh=plsc.VectorSubcoreMesh(**_MESH),
    scratch_types=[
        pltpu.VMEM((NCHUNK, CHUNK), jnp.int32),
        pltpu.VMEM((CHUNK, DEGW), jnp.float32),
        pltpu.VMEM_SHARED((NPAD, DEGW), jnp.float32),
        pltpu.SemaphoreType.DMA((NBUF,)),
        pltpu.SemaphoreType.DMA,
    ],
)
def _sc_degree(ei_hbm, out_hbm, didx_v, ones_v, acc_sh, ssem, psem):
    c = lax.axis_index("c")
    s = lax.axis_index("s")
    wid = c * NS + s

    _zero_vmem_2d(ones_v, CHUNK, DEGW)

    def zinit(j, _):
        pltpu.sync_copy(ones_v, acc_sh.at[pl.ds(s * RPS + j * CHUNK, CHUNK)])
        return _

    lax.fori_loop(0, RPS // CHUNK, zinit, None)

    one = jnp.full((L,), 1.0, jnp.float32)

    def fill(i, _):
        ones_v[i, :] = one
        return _

    lax.fori_loop(0, CHUNK, fill, None)

    def pre(i, _):
        base = E + wid * EPW + i * CHUNK
        pltpu.make_async_copy(ei_hbm.at[pl.ds(base, CHUNK)], didx_v.at[i],
                              psem).start()
        return _

    def pre_wait(i, _):
        pltpu.make_async_copy(ei_hbm.at[pl.ds(E + wid * EPW, CHUNK)],
                              didx_v.at[0], psem).wait()
        return _

    lax.fori_loop(0, NCHUNK, pre, None)
    lax.fori_loop(0, NCHUNK, pre_wait, None)
    plsc.subcore_barrier()

    def scat_wait(b):
        pltpu.make_async_copy(ones_v, acc_sh.at[didx_v.at[0]],
                              ssem.at[b]).wait()

    def step(i, _):
        b = lax.rem(i, NBUF)

        @pl.when(i >= NBUF)
        def _():
            scat_wait(b)

        pltpu.async_copy(ones_v, acc_sh.at[didx_v.at[i]], ssem.at[b],
                         add=True)
        return _

    lax.fori_loop(0, NCHUNK, step, None)
    for t in range(NBUF):
        scat_wait((NCHUNK - 1 - t) % NBUF)
    plsc.subcore_barrier()

    def wb(j, _):
        r0 = s * RPS + j * CHUNK
        pltpu.sync_copy(acc_sh.at[pl.ds(r0, CHUNK)], ones_v)
        pltpu.sync_copy(ones_v, out_hbm.at[c, pl.ds(r0, CHUNK)])
        return _

    lax.fori_loop(0, RPS // CHUNK, wb, None)


_RB = 2000


def _prep_body(x_ref, w_ref, dp_ref, g_ref):
    h = jnp.dot(x_ref[...], w_ref[...], preferred_element_type=jnp.float32)
    deg = dp_ref[0, :, 0:1] + dp_ref[1, :, 0:1] + 1.0
    g_ref[...] = h * lax.rsqrt(deg)


def _tc_prep(x, W, degpart):
    return pl.pallas_call(
        _prep_body,
        grid=(N // _RB,),
        in_specs=[
            pl.BlockSpec((_RB, D), lambda i: (i, 0)),
            pl.BlockSpec((D, D), lambda i: (0, 0)),
            pl.BlockSpec((NC, _RB, DEGW), lambda i: (0, i, 0)),
        ],
        out_specs=pl.BlockSpec((_RB, D), lambda i: (i, 0)),
        out_shape=jax.ShapeDtypeStruct((N, D), jnp.float32),
    )(x, W, degpart)


@functools.partial(
    pl.kernel,
    out_type=jax.ShapeDtypeStruct((NC, NPAD, D), jnp.float32),
    mesh=plsc.VectorSubcoreMesh(**_MESH),
    scratch_types=[
        pltpu.VMEM((EPW,), jnp.int32),
        pltpu.VMEM((NBUF, CHUNK), jnp.int32),
        pltpu.VMEM((NBUF, CHUNK, D), jnp.float32),
        pltpu.VMEM_SHARED((NPAD, D), jnp.float32),
        pltpu.SemaphoreType.DMA((NBUF,)),
        pltpu.SemaphoreType.DMA((NBUF,)),
        pltpu.SemaphoreType.DMA((NBUF,)),
    ],
)
def _sc_edges(g_hbm, ei_hbm, out_hbm, sidx_v, didx_v, rows_v,
              acc_sh, dsem, gsem, ssem):
    c = lax.axis_index("c")
    s = lax.axis_index("s")
    wid = c * NS + s

    _zero_vmem_2d(rows_v.at[0], CHUNK, D)

    def zinit(j, _):
        pltpu.sync_copy(rows_v.at[0],
                        acc_sh.at[pl.ds(s * RPS + j * CHUNK, CHUNK)])
        return _

    lax.fori_loop(0, RPS // CHUNK, zinit, None)
    pltpu.sync_copy(ei_hbm.at[pl.ds(wid * EPW, EPW)], sidx_v)
    plsc.subcore_barrier()

    def didx_copy(i, b):
        base = E + wid * EPW + i * CHUNK
        return pltpu.make_async_copy(ei_hbm.at[pl.ds(base, CHUNK)],
                                     didx_v.at[b], dsem.at[b])

    def gath(i, b):
        return pltpu.make_async_copy(
            g_hbm.at[sidx_v.at[pl.ds(i * CHUNK, CHUNK)]], rows_v.at[b],
            gsem.at[b])

    def scat_wait(b):
        pltpu.make_async_copy(rows_v.at[b], acc_sh.at[didx_v.at[b]],
                              ssem.at[b]).wait()

    didx_copy(0, 0).start()
    gath(0, 0).start()

    def step(i, _):
        b = lax.rem(i, NBUF)
        b1 = lax.rem(i + 1, NBUF)

        @pl.when(i + 1 < NCHUNK)
        def _():
            @pl.when(i + 1 >= NBUF)
            def _():
                scat_wait(b1)
            didx_copy(i + 1, b1).start()
            gath(i + 1, b1).start()

        didx_copy(i, b).wait()
        gath(i, b).wait()
        pltpu.async_copy(rows_v.at[b], acc_sh.at[didx_v.at[b]], ssem.at[b],
                         add=True)
        return _

    lax.fori_loop(0, NCHUNK, step, None)
    for t in range(NBUF):
        scat_wait((NCHUNK - 1 - t) % NBUF)
    plsc.subcore_barrier()

    def wb(j, _):
        r0 = s * RPS + j * CHUNK
        pltpu.sync_copy(acc_sh.at[pl.ds(r0, CHUNK)], rows_v.at[0])
        pltpu.sync_copy(rows_v.at[0], out_hbm.at[c, pl.ds(r0, CHUNK)])
        return _

    lax.fori_loop(0, RPS // CHUNK, wb, None)


def _final_body(ap_ref, g_ref, dp_ref, b_ref, a_ref, o_ref):
    deg = dp_ref[0, :, 0:1] + dp_ref[1, :, 0:1] + 1.0
    dinv = lax.rsqrt(deg)
    z = dinv * (ap_ref[0] + ap_ref[1] + g_ref[...]) + b_ref[...]
    o_ref[...] = jnp.where(z >= 0, z, a_ref[0, 0] * z)


def _tc_final(accpart, g, degpart, b2, a2):
    return pl.pallas_call(
        _final_body,
        grid=(N // _RB,),
        in_specs=[
            pl.BlockSpec((NC, _RB, D), lambda i: (0, i, 0)),
            pl.BlockSpec((_RB, D), lambda i: (i, 0)),
            pl.BlockSpec((NC, _RB, DEGW), lambda i: (0, i, 0)),
            pl.BlockSpec((1, D), lambda i: (0, 0)),
            pl.BlockSpec((1, 1), lambda i: (0, 0)),
        ],
        out_specs=pl.BlockSpec((_RB, D), lambda i: (i, 0)),
        out_shape=jax.ShapeDtypeStruct((N, D), jnp.float32),
    )(accpart, g, degpart, b2, a2)


def kernel(x, edge_index, W, b, prelu_a):
    eiflat = edge_index.astype(jnp.int32).reshape(2 * E)
    degpart = _sc_degree(eiflat)
    g = _tc_prep(x, W, degpart)
    accpart = _sc_edges(g, eiflat)
    return _tc_final(accpart, g, degpart, b.reshape(1, D),
                     prelu_a.reshape(1, 1))
